# baseline (speedup 1.0000x reference)
import functools

import jax
import jax.numpy as jnp
from jax import lax
from jax.experimental import pallas as pl
from jax.experimental.pallas import tpu as pltpu
from jax.experimental.pallas import tpu_sc as plsc

_NC = 2
_NS = 16
_NW = _NC * _NS

_BATCH = 16384
_EMBED = 64
_IDX_CHUNK = 128
_B_PER_W = _BATCH // _NW
_CHUNKS_PER_W = _B_PER_W // _IDX_CHUNK


def _gather_body(user_idx_hbm, movie_idx_hbm, user_table_hbm, movie_table_hbm,
                 uvec_hbm, mvec_hbm, idx_u, idx_m, rows_u, rows_m, sem_u,
                 sem_m, sem_i):
  wid = lax.axis_index("s") * _NC + lax.axis_index("c")
  base = wid * _B_PER_W
  row0 = wid * _CHUNKS_PER_W
  cu = pltpu.async_copy(user_idx_hbm.at[pl.ds(row0, _CHUNKS_PER_W)], idx_u,
                        sem_i)
  cm = pltpu.async_copy(movie_idx_hbm.at[pl.ds(row0, _CHUNKS_PER_W)], idx_m,
                        sem_i)
  cu.wait()
  cm.wait()
  copies = []
  for j in range(_CHUNKS_PER_W):
    dst = rows_u.at[pl.ds(j * _IDX_CHUNK, _IDX_CHUNK)]
    copies.append(
        pltpu.async_copy(user_table_hbm.at[idx_u.at[j]], dst, sem_u))
  for j in range(_CHUNKS_PER_W):
    dst = rows_m.at[pl.ds(j * _IDX_CHUNK, _IDX_CHUNK)]
    copies.append(
        pltpu.async_copy(movie_table_hbm.at[idx_m.at[j]], dst, sem_m))
  for c in copies:
    c.wait()
  pltpu.sync_copy(rows_u, uvec_hbm.at[pl.ds(base, _B_PER_W)])
  pltpu.sync_copy(rows_m, mvec_hbm.at[pl.ds(base, _B_PER_W)])


@jax.jit
def _sc_gather(user, movie, user_table, movie_table):
  user2d = user.reshape(_BATCH // _IDX_CHUNK, _IDX_CHUNK)
  movie2d = movie.reshape(_BATCH // _IDX_CHUNK, _IDX_CHUNK)
  mesh = plsc.VectorSubcoreMesh(core_axis_name="c", subcore_axis_name="s")
  fn = pl.kernel(
      _gather_body,
      out_type=[
          jax.ShapeDtypeStruct((_BATCH, _EMBED), jnp.float32),
          jax.ShapeDtypeStruct((_BATCH, _EMBED), jnp.float32),
      ],
      mesh=mesh,
      compiler_params=pltpu.CompilerParams(use_tc_tiling_on_sc=False),
      scratch_types=[
          pltpu.VMEM((_CHUNKS_PER_W, _IDX_CHUNK), jnp.int32),
          pltpu.VMEM((_CHUNKS_PER_W, _IDX_CHUNK), jnp.int32),
          pltpu.VMEM((_B_PER_W, _EMBED), jnp.float32),
          pltpu.VMEM((_B_PER_W, _EMBED), jnp.float32),
          pltpu.SemaphoreType.DMA,
          pltpu.SemaphoreType.DMA,
          pltpu.SemaphoreType.DMA,
      ],
  )
  return fn(user2d, movie2d, user_table, movie_table)


_BLK = 2048


def _mlp_body(u_ref, m_ref, w1u_ref, w1m_ref, b1_ref, w2t_ref, b2_ref,
              out_ref):
  h = (jnp.dot(u_ref[:], w1u_ref[:], preferred_element_type=jnp.float32) +
       jnp.dot(m_ref[:], w1m_ref[:], preferred_element_type=jnp.float32) +
       b1_ref[:])
  h = jnp.maximum(h, 0.0)
  out_ref[:] = (jnp.sum(h * w2t_ref[:], axis=1, keepdims=True) + b2_ref[:])


@jax.jit
def _tc_mlp(uvec, mvec, W1, b1, W2, b2):
  w1u = W1[:_EMBED]
  w1m = W1[_EMBED:]
  b1r = b1.reshape(1, 128)
  w2t = W2.reshape(1, 128)
  b2r = b2.reshape(1, 1)
  grid = (_BATCH // _BLK,)
  return pl.pallas_call(
      _mlp_body,
      grid=grid,
      in_specs=[
          pl.BlockSpec((_BLK, _EMBED), lambda i: (i, 0)),
          pl.BlockSpec((_BLK, _EMBED), lambda i: (i, 0)),
          pl.BlockSpec((_EMBED, 128), lambda i: (0, 0)),
          pl.BlockSpec((_EMBED, 128), lambda i: (0, 0)),
          pl.BlockSpec((1, 128), lambda i: (0, 0)),
          pl.BlockSpec((1, 128), lambda i: (0, 0)),
          pl.BlockSpec((1, 1), lambda i: (0, 0)),
      ],
      out_specs=pl.BlockSpec((_BLK, 1), lambda i: (i, 0)),
      out_shape=jax.ShapeDtypeStruct((_BATCH, 1), jnp.float32),
  )(uvec, mvec, w1u, w1m, b1r, w2t, b2r)


def kernel(user, movie, user_table, movie_table, W1, b1, W2, b2):
  uvec, mvec = _sc_gather(user, movie, user_table, movie_table)
  return _tc_mlp(uvec, mvec, W1, b1, W2, b2)
